# baseline (speedup 1.0000x reference)
import functools

import jax
import jax.numpy as jnp
from jax import lax
from jax.experimental import pallas as pl
from jax.experimental.pallas import tpu as pltpu
from jax.experimental.pallas import tpu_sc as plsc

_VOCAB = 32002
_HIDDEN = 2048
_BATCH = 4
_SEQ = 4096
_NTOK = _BATCH * _SEQ
_NW = 32
_PER_W = _NTOK // _NW
_CHUNK = 8
_NCHUNK = _PER_W // _CHUNK
_NGROUP = _NCHUNK // 4

_mesh = plsc.VectorSubcoreMesh(core_axis_name="c", subcore_axis_name="s")


@functools.partial(
    pl.kernel,
    out_type=jax.ShapeDtypeStruct((_NTOK, _HIDDEN), jnp.float32),
    mesh=_mesh,
    scratch_types=(
        [pltpu.VMEM((_NCHUNK, _CHUNK), jnp.int32)]
        + [pltpu.VMEM((_CHUNK, _HIDDEN), jnp.float32)] * 4
        + [pltpu.VMEM_SHARED((16, 3, _CHUNK, _HIDDEN), jnp.float32)]
        + [pltpu.SemaphoreType.DMA] * 4
        + [pltpu.SemaphoreType.DMA] * 3
        + [pltpu.SemaphoreType.DMA] * 3
        + [pltpu.SemaphoreType.DMA]
    ),
)
def _embed_lookup(table_hbm, idx_hbm, out_hbm, idx_v, *scratch):
    sid = lax.axis_index("s")
    wid = sid * 2 + lax.axis_index("c")
    base = wid * _PER_W
    pltpu.sync_copy(idx_hbm.at[wid], idx_v)

    bufs = scratch[:4]
    shared = scratch[4]
    gsems = scratch[5:9]
    xsems = scratch[9:12]
    osems = scratch[12:15]
    dsem = scratch[15]

    def gather_start(j, b):
        pltpu.async_copy(table_hbm.at[idx_v.at[j]], bufs[b], gsems[b])

    def gather_wait(b):
        pltpu.make_async_copy(
            table_hbm.at[idx_v.at[0]], bufs[b], gsems[b]).wait()

    def stage(i):
        pltpu.async_copy(bufs[i], shared.at[sid, i], xsems[i]).wait()

    def out_start(j, i):
        pltpu.async_copy(
            shared.at[sid, i],
            out_hbm.at[pl.ds(base + j * _CHUNK, _CHUNK)], osems[i])

    def out_wait(i):
        pltpu.make_async_copy(
            shared.at[sid, i],
            out_hbm.at[pl.ds(base, _CHUNK)], osems[i]).wait()

    def dstart(j):
        pltpu.async_copy(
            bufs[3], out_hbm.at[pl.ds(base + j * _CHUNK, _CHUNK)], dsem)

    def dwait():
        pltpu.make_async_copy(
            bufs[3], out_hbm.at[pl.ds(base, _CHUNK)], dsem).wait()

    def group_body(k, first=False, prefetch=True):
        jg = 4 * k
        for i in range(3):
            j = jg + i
            if not first:
                out_wait(i)
            gather_wait(i)
            stage(i)
            out_start(j, i)
            if prefetch:
                gather_start(j + 4, i)
        if not first:
            dwait()
        gather_start(jg + 3, 3)
        gather_wait(3)
        dstart(jg + 3)

    for i in range(3):
        gather_start(i, i)
    group_body(0, first=True)

    def step(k, carry):
        group_body(k)
        return carry

    lax.fori_loop(1, _NGROUP - 1, step, 0)

    group_body(_NGROUP - 1, prefetch=False)
    for i in range(3):
        out_wait(i)
    dwait()


def kernel(input_ids, embedding_table):
    ids = input_ids.reshape(_NW, _NCHUNK, _CHUNK)
    out = _embed_lookup(embedding_table, ids)
    return out.reshape(_BATCH, _SEQ, _HIDDEN)


# speedup vs baseline: 1.0356x; 1.0356x over previous
import functools

import jax
import jax.numpy as jnp
from jax import lax
from jax.experimental import pallas as pl
from jax.experimental.pallas import tpu as pltpu
from jax.experimental.pallas import tpu_sc as plsc

_VOCAB = 32002
_HIDDEN = 2048
_BATCH = 4
_SEQ = 4096
_NTOK = _BATCH * _SEQ
_NW = 32
_PER_W = _NTOK // _NW
_CHUNK = 8
_NCHUNK = _PER_W // _CHUNK
_NBUF = 3

_mesh = plsc.VectorSubcoreMesh(core_axis_name="c", subcore_axis_name="s")


@functools.partial(
    pl.kernel,
    out_type=jax.ShapeDtypeStruct((_NTOK, _HIDDEN), jnp.float32),
    mesh=_mesh,
    scratch_types=(
        [pltpu.VMEM((_NCHUNK, _CHUNK), jnp.int32)]
        + [pltpu.VMEM((_CHUNK, _HIDDEN), jnp.float32)] * _NBUF
        + [pltpu.VMEM_SHARED((16, _NBUF, _CHUNK, _HIDDEN), jnp.float32)]
        + [pltpu.SemaphoreType.DMA] * (3 * _NBUF)
    ),
)
def _embed_lookup(table_hbm, idx_hbm, out_hbm, idx_v, *scratch):
    sid = lax.axis_index("s")
    wid = sid * 2 + lax.axis_index("c")
    base = wid * _PER_W
    pltpu.sync_copy(idx_hbm.at[wid], idx_v)

    bufs = scratch[:_NBUF]
    shared = scratch[_NBUF]
    gsems = scratch[_NBUF + 1:2 * _NBUF + 1]
    xsems = scratch[2 * _NBUF + 1:3 * _NBUF + 1]
    osems = scratch[3 * _NBUF + 1:]

    def gather_start(j, b):
        pltpu.async_copy(table_hbm.at[idx_v.at[j]], bufs[b], gsems[b])

    def gather_wait(b):
        pltpu.make_async_copy(
            table_hbm.at[idx_v.at[0]], bufs[b], gsems[b]).wait()

    def stage(b):
        pltpu.async_copy(bufs[b], shared.at[sid, b], xsems[b]).wait()

    def out_start(j, b):
        pltpu.async_copy(
            shared.at[sid, b],
            out_hbm.at[pl.ds(base + j * _CHUNK, _CHUNK)], osems[b])

    def out_wait(b):
        pltpu.make_async_copy(
            shared.at[sid, b],
            out_hbm.at[pl.ds(base, _CHUNK)], osems[b]).wait()

    def slot_body(j, b, first=False, last=False):
        if not first:
            out_wait(b)
        gather_wait(b)
        stage(b)
        out_start(j, b)
        if not last:
            gather_start(j + _NBUF, b)

    for b in range(_NBUF):
        gather_start(b, b)
    for b in range(_NBUF):
        slot_body(b, b, first=True)

    def step(k, carry):
        for p in range(_NBUF):
            slot_body(_NBUF * k + p, p)
        return carry

    _KMAX = (_NCHUNK - _NBUF) // _NBUF - 1
    lax.fori_loop(1, _KMAX + 1, step, 0)

    _TAIL_START = _NBUF * (_KMAX + 1)
    slot_body(_TAIL_START, _TAIL_START % _NBUF)
    for j in range(_TAIL_START + 1, _NCHUNK):
        slot_body(j, j % _NBUF, last=True)
    for j in range(_NCHUNK - _NBUF, _NCHUNK):
        out_wait(j % _NBUF)


def kernel(input_ids, embedding_table):
    ids = input_ids.reshape(_NW, _NCHUNK, _CHUNK)
    out = _embed_lookup(embedding_table, ids)
    return out.reshape(_BATCH, _SEQ, _HIDDEN)
